# baseline (device time: 9875 ns/iter reference)
import jax
import jax.numpy as jnp
from jax import lax
from jax.experimental import pallas as pl
from jax.experimental.pallas import tpu as pltpu

N_DEV = 16


def kernel(A, B):
    m, k = A.shape
    k2, n = B.shape
    rows = m // N_DEV

    def body(a_ref, b_ref, out_ref, send16):
        d = lax.axis_index("i")

        def chunk(c):
            return pl.ds(lax.rem(c + 2 * N_DEV, N_DEV) * rows, rows)

        barrier_sem = pltpu.get_barrier_semaphore()
        for off in range(1, N_DEV):
            pl.semaphore_signal(
                barrier_sem, inc=1,
                device_id=(lax.rem(d + off, N_DEV),),
                device_id_type=pl.DeviceIdType.MESH,
            )
        pl.semaphore_wait(barrier_sem, N_DEV - 1)

        out_ref[:, :] = jnp.dot(
            a_ref[:, :].astype(jnp.bfloat16),
            b_ref[:, :].astype(jnp.bfloat16),
            preferred_element_type=jnp.float32,
        )
        for off in range(1, N_DEV):
            o = lax.rem(d + off, N_DEV)
            send16[off] = out_ref[chunk(o)].astype(jnp.bfloat16)

        acc = out_ref[chunk(d)]
        z = acc / (1.0 + jnp.exp(-acc))
        out_ref[chunk(d)] = z
        send16[0] = z.astype(jnp.bfloat16)

    return pl.pallas_call(
        body,
        out_shape=jax.ShapeDtypeStruct((m, n), jnp.float32),
        in_specs=[
            pl.BlockSpec(memory_space=pltpu.VMEM),
            pl.BlockSpec(memory_space=pltpu.VMEM),
        ],
        out_specs=pl.BlockSpec(memory_space=pltpu.VMEM),
        scratch_shapes=[
            pltpu.VMEM((N_DEV, m // N_DEV, n), jnp.bfloat16),
        ],
        compiler_params=pltpu.CompilerParams(collective_id=0),
    )(A, B)


# device time: 3623 ns/iter; 2.7256x vs baseline; 2.7256x over previous
import jax
import jax.numpy as jnp
from jax import lax
from jax.experimental import pallas as pl
from jax.experimental.pallas import tpu as pltpu

N_DEV = 16


def kernel(A, B):
    m, k = A.shape
    k2, n = B.shape
    rows = m // N_DEV

    def body(a_ref, b_ref, out_ref, send16):
        d = lax.axis_index("i")

        def chunk(c):
            return pl.ds(lax.rem(c + 2 * N_DEV, N_DEV) * rows, rows)

        out_ref[:, :] = jnp.dot(
            a_ref[:, :].astype(jnp.bfloat16),
            b_ref[:, :].astype(jnp.bfloat16),
            preferred_element_type=jnp.float32,
        )
        for off in range(1, N_DEV):
            o = lax.rem(d + off, N_DEV)
            send16[off] = out_ref[chunk(o)].astype(jnp.bfloat16)

        acc = out_ref[chunk(d)]
        z = acc / (1.0 + jnp.exp(-acc))
        out_ref[chunk(d)] = z
        send16[0] = z.astype(jnp.bfloat16)

    return pl.pallas_call(
        body,
        out_shape=jax.ShapeDtypeStruct((m, n), jnp.float32),
        in_specs=[
            pl.BlockSpec(memory_space=pltpu.VMEM),
            pl.BlockSpec(memory_space=pltpu.VMEM),
        ],
        out_specs=pl.BlockSpec(memory_space=pltpu.VMEM),
        scratch_shapes=[
            pltpu.VMEM((N_DEV, m // N_DEV, n), jnp.bfloat16),
        ],
    )(A, B)
